# Initial kernel scaffold; baseline (speedup 1.0000x reference)
import jax
import jax.numpy as jnp
from jax.experimental import pallas as pl


def kernel(feat, edge_index, W, b, prelu_a):
    raise NotImplementedError("write your pallas kernel here")


# speedup vs baseline: 1.5008x; 1.5008x over previous
import jax
import jax.numpy as jnp
from jax import lax
from jax.experimental import pallas as pl
from jax.experimental.pallas import tpu as pltpu
from jax.experimental.pallas import tpu_sc as plsc

N = 10000
E = 320000
D = 128
KSTEPS = 10
ALPHA = 0.1

NT = 16
NC = 2
DH = D // NC
PW = D
NP = 10240
NPT = NP // NT
NARR = NPT + 16
ECH = 128
EPT = 20480
EP = NT * EPT
NCH = EPT // ECH
NRC = 64
NNC = NPT // NRC


def _rsqrt16(x):
    i = plsc.bitcast(x, jnp.int32)
    y = plsc.bitcast(jnp.int32(0x5F3759DF) - (i >> 1), jnp.float32)
    for _ in range(3):
        y = y * (1.5 - 0.5 * x * y * y)
    return y


def _sc_body(feat_hbm, eidx_hbm, hk_hbm, c0_hbm, g_hbm,
             acc, degall, gidx, sidx, counts, buf, tmp,
             no_a, ni_a, a_a, sem):
    cid = lax.axis_index("c")
    sid = lax.axis_index("s")
    ebase = sid * EPT
    nbase = sid * NPT
    gplane = cid * NP
    coff16 = jnp.full((16,), gplane, jnp.int32)

    zero16 = jnp.zeros((16,), jnp.float32)
    one16 = jnp.ones((16,), jnp.float32)

    for rnd, dacc in ((0, no_a), (1, ni_a)):
        def z_counts(i, _):
            counts[pl.ds(i * 16, 16)] = zero16
            return 0
        lax.fori_loop(0, NP // 16, z_counts, 0)

        def cnt(j, _, rnd=rnd):
            pltpu.sync_copy(
                eidx_hbm.at[pl.ds(rnd * EP + ebase + j * ECH, ECH)], gidx)
            for k in range(ECH // 16):
                ids = gidx[pl.ds(k * 16, 16)]
                plsc.addupdate_scatter(counts, [ids], one16)
            return 0
        lax.fori_loop(0, NCH, cnt, 0)

        pltpu.sync_copy(counts, degall.at[pl.ds(sid * NP, NP)])
        plsc.subcore_barrier()

        def z_deg(i, _, dacc=dacc):
            dacc[pl.ds(i * 16, 16)] = zero16
            return 0
        lax.fori_loop(0, NARR // 16, z_deg, 0)

        def merge(t2, _, dacc=dacc):
            pltpu.sync_copy(degall.at[pl.ds(t2 * NP + nbase, NPT)], tmp)

            def acc_l(i, _):
                dacc[pl.ds(i * 16, 16)] += tmp[pl.ds(i * 16, 16)]
                return 0
            lax.fori_loop(0, NPT // 16, acc_l, 0)
            return 0
        lax.fori_loop(0, NT, merge, 0)
        plsc.subcore_barrier()

    def mk_norm(i, _):
        do = jnp.maximum(no_a[pl.ds(i * 16, 16)], 1.0)
        di = jnp.maximum(ni_a[pl.ds(i * 16, 16)], 1.0)
        no = _rsqrt16(do)
        ni = _rsqrt16(di)
        no_a[pl.ds(i * 16, 16)] = no
        ni_a[pl.ds(i * 16, 16)] = ni
        a_a[pl.ds(i * 16, 16)] = (1.0 - ALPHA) * no * ni
        return 0
    lax.fori_loop(0, NARR // 16, mk_norm, 0)

    def _zero_edge_rows():
        def zr(r, _):
            for v in range(PW // 16):
                buf[r, pl.ds(v * 16, 16)] = zero16
            return 0
        lax.fori_loop(0, NRC, zr, 0)

    def init_node(q, _):
        r0 = nbase + q * NRC
        pltpu.sync_copy(feat_hbm.at[cid, pl.ds(r0, NRC)],
                        buf.at[pl.ds(NRC, NRC)])

        def initrow(r, _):
            nov = jnp.full((16,), no_a[pl.ds(q * NRC + r, 16)][0],
                           jnp.float32)
            for v in range(PW // 16):
                g0 = nov * buf[NRC + r, pl.ds(v * 16, 16)]
                buf[r, pl.ds(v * 16, 16)] = g0
                buf[NRC + r, pl.ds(v * 16, 16)] = ALPHA * g0
            return 0
        lax.fori_loop(0, NRC, initrow, 0)
        pltpu.sync_copy(buf.at[pl.ds(0, NRC)],
                        g_hbm.at[pl.ds(gplane + r0, NRC)])
        pltpu.sync_copy(buf.at[pl.ds(NRC, NRC)],
                        c0_hbm.at[cid, pl.ds(r0, NRC)])
        _zero_edge_rows()
        pltpu.sync_copy(buf.at[pl.ds(0, NRC)], acc.at[pl.ds(r0, NRC)])
        return 0
    lax.fori_loop(0, NNC, init_node, 0)
    plsc.subcore_barrier()

    NCNP = NC * NP

    def do_edges(par):
        offv = jnp.full((16,), par * NCNP + gplane, jnp.int32)

        def edge(j, _):
            pltpu.sync_copy(eidx_hbm.at[pl.ds(ebase + j * ECH, ECH)], gidx)
            pltpu.sync_copy(eidx_hbm.at[pl.ds(EP + ebase + j * ECH, ECH)],
                            sidx)
            for k in range(ECH // 16):
                gidx[pl.ds(k * 16, 16)] += offv
            pltpu.async_copy(g_hbm.at[gidx], buf.at[pl.ds(0, ECH)],
                             sem).wait()
            pltpu.sync_copy(buf.at[pl.ds(0, ECH)], acc.at[sidx], add=True)
            return 0
        lax.fori_loop(0, NCH, edge, 0)

    def step_body(t, _):
        par = lax.rem(t, 2)
        do_edges(par)
        plsc.subcore_barrier()

        def node(q, _):
            r0 = nbase + q * NRC
            pltpu.sync_copy(acc.at[pl.ds(r0, NRC)], buf.at[pl.ds(0, NRC)])
            pltpu.sync_copy(c0_hbm.at[cid, pl.ds(r0, NRC)],
                            buf.at[pl.ds(NRC, NRC)])

            def uprow(r, _):
                sa = jnp.full((16,), a_a[pl.ds(q * NRC + r, 16)][0],
                              jnp.float32)
                for v in range(PW // 16):
                    gv = sa * buf[r, pl.ds(v * 16, 16)]
                    gv = gv + buf[NRC + r, pl.ds(v * 16, 16)]
                    buf[r, pl.ds(v * 16, 16)] = gv
                return 0
            lax.fori_loop(0, NRC, uprow, 0)
            pltpu.sync_copy(
                buf.at[pl.ds(0, NRC)],
                g_hbm.at[pl.ds((1 - par) * NCNP + gplane + r0, NRC)])
            _zero_edge_rows()
            pltpu.sync_copy(buf.at[pl.ds(0, NRC)], acc.at[pl.ds(r0, NRC)])
            return 0
        lax.fori_loop(0, NNC, node, 0)
        plsc.subcore_barrier()
        return 0
    lax.fori_loop(0, KSTEPS - 1, step_body, 0)

    do_edges(jnp.int32((KSTEPS - 1) % 2))
    plsc.subcore_barrier()

    def node_last(q, _):
        r0 = nbase + q * NRC
        pltpu.sync_copy(acc.at[pl.ds(r0, NRC)], buf.at[pl.ds(0, NRC)])
        pltpu.sync_copy(feat_hbm.at[cid, pl.ds(r0, NRC)],
                        buf.at[pl.ds(NRC, NRC)])

        def uprow(r, _):
            sa = jnp.full((16,),
                          (1.0 - ALPHA) * ni_a[pl.ds(q * NRC + r, 16)][0],
                          jnp.float32)
            for v in range(PW // 16):
                gv = sa * buf[r, pl.ds(v * 16, 16)]
                gv = gv + ALPHA * buf[NRC + r, pl.ds(v * 16, 16)]
                buf[r, pl.ds(v * 16, 16)] = gv
            return 0
        lax.fori_loop(0, NRC, uprow, 0)
        pltpu.sync_copy(buf.at[pl.ds(0, NRC)],
                        hk_hbm.at[cid, pl.ds(r0, NRC)])
        return 0
    lax.fori_loop(0, NNC, node_last, 0)


_sc_propagate = pl.kernel(
    _sc_body,
    out_type=(jax.ShapeDtypeStruct((NC, NP, PW), jnp.float32),
              jax.ShapeDtypeStruct((NC, NP, PW), jnp.float32),
              jax.ShapeDtypeStruct((2 * NC * NP, PW), jnp.float32)),
    mesh=plsc.VectorSubcoreMesh(core_axis_name="c", subcore_axis_name="s"),
    compiler_params=pltpu.CompilerParams(needs_layout_passes=False),
    scratch_types=[
        pltpu.VMEM_SHARED((NP, PW), jnp.float32),
        pltpu.VMEM_SHARED((NT * NP,), jnp.float32),
        pltpu.VMEM((ECH,), jnp.int32),
        pltpu.VMEM((ECH,), jnp.int32),
        pltpu.VMEM((NP,), jnp.float32),
        pltpu.VMEM((2 * NRC, PW), jnp.float32),
        pltpu.VMEM((NPT,), jnp.float32),
        pltpu.VMEM((NARR,), jnp.float32),
        pltpu.VMEM((NARR,), jnp.float32),
        pltpu.VMEM((NARR,), jnp.float32),
        pltpu.SemaphoreType.DMA,
    ],
)


BN = 1000


def _tc_body(h_ref, w_ref, b_ref, a_ref, out_ref, gs_ref):
    i = pl.program_id(0)
    x = h_ref[...]
    y = lax.dot_general(x, w_ref[...], (((1,), (1,)), ((), ())),
                        preferred_element_type=jnp.float32)
    y = y + b_ref[...]
    a = a_ref[0, 0]
    y = jnp.where(y >= 0.0, y, a * y)
    out_ref[...] = y
    part = jnp.sum(y, axis=0, keepdims=True)

    @pl.when(i == 0)
    def _():
        gs_ref[...] = jnp.zeros((1, D), jnp.float32)
    gs_ref[...] += part


def _tc_head(hk, W, b, prelu_a):
    return pl.pallas_call(
        _tc_body,
        grid=(N // BN,),
        in_specs=[pl.BlockSpec((BN, D), lambda i: (i, 0)),
                  pl.BlockSpec((D, D), lambda i: (0, 0)),
                  pl.BlockSpec((1, D), lambda i: (0, 0)),
                  pl.BlockSpec((1, 1), lambda i: (0, 0))],
        out_specs=[pl.BlockSpec((BN, D), lambda i: (i, 0)),
                   pl.BlockSpec((1, D), lambda i: (0, 0))],
        out_shape=[jax.ShapeDtypeStruct((N, D), jnp.float32),
                   jax.ShapeDtypeStruct((1, D), jnp.float32)],
    )(hk, W, b.reshape(1, D), prelu_a.reshape(1, 1))


def kernel(feat, edge_index, W, b, prelu_a):
    feat_p = jnp.pad(feat, ((0, NP - N), (0, 0)))
    planes = [
        jnp.pad(feat_p[:, c * DH:(c + 1) * DH], ((0, 0), (0, PW - DH)))
        for c in range(NC)
    ]
    feat_cols = jnp.stack(planes)
    eidx_p = jnp.pad(edge_index, ((0, 0), (0, EP - E)), constant_values=N)
    hk_cols, _, _ = _sc_propagate(feat_cols, eidx_p.reshape(-1))
    hk = jnp.concatenate([hk_cols[c, :N, :DH] for c in range(NC)], axis=1)
    out, gsum = _tc_head(hk, W, b, prelu_a)
    return (out, gsum)


# speedup vs baseline: 1.8437x; 1.2285x over previous
import jax
import jax.numpy as jnp
from jax import lax
from jax.experimental import pallas as pl
from jax.experimental.pallas import tpu as pltpu
from jax.experimental.pallas import tpu_sc as plsc

N = 10000
E = 320000
D = 128
KSTEPS = 10
ALPHA = 0.1

NT = 16
NC = 2
DH = D // NC
PW = D
NP = 10240
NPT = NP // NT
NARR = NPT + 16
ECH = 128
EPT = 20480
EP = NT * EPT
NCH = EPT // ECH
NRC = 64
NNC = NPT // NRC


def _rsqrt16(x):
    i = plsc.bitcast(x, jnp.int32)
    y = plsc.bitcast(jnp.int32(0x5F3759DF) - (i >> 1), jnp.float32)
    for _ in range(3):
        y = y * (1.5 - 0.5 * x * y * y)
    return y


def _sc_body(feat_hbm, eidx_hbm, hk_hbm, c0_hbm, g_hbm, degall,
             acc, gidx_a, gidx_b, sidx_a, sidx_b, counts,
             buf, buf_b, tmp, no_a, ni_a, a_a,
             gsem_a, gsem_b, ssem):
    cid = lax.axis_index("c")
    sid = lax.axis_index("s")
    ebase = sid * EPT
    nbase = sid * NPT
    gplane = cid * NP
    coff16 = jnp.full((16,), gplane, jnp.int32)

    zero16 = jnp.zeros((16,), jnp.float32)
    one16 = jnp.ones((16,), jnp.float32)

    if True:
        for rnd, dacc in ((0, no_a), (1, ni_a)):
            def z_counts(i, _):
                counts[pl.ds(i * 16, 16)] = zero16
                return 0
            lax.fori_loop(0, NP // 16, z_counts, 0)

            def cnt(j, _, rnd=rnd):
                pltpu.sync_copy(
                    eidx_hbm.at[pl.ds(rnd * EP + ebase + j * ECH, ECH)],
                    gidx_a)
                for k in range(ECH // 16):
                    ids = gidx_a[pl.ds(k * 16, 16)]
                    plsc.addupdate_scatter(counts, [ids], one16)
                return 0
            lax.fori_loop(0, NCH, cnt, 0)

            pltpu.sync_copy(counts, degall.at[cid, pl.ds(sid * NP, NP)])
            plsc.subcore_barrier()

            def z_deg(i, _, dacc=dacc):
                dacc[pl.ds(i * 16, 16)] = zero16
                return 0
            lax.fori_loop(0, NARR // 16, z_deg, 0)

            def merge(t2, _, dacc=dacc):
                pltpu.sync_copy(degall.at[cid, pl.ds(t2 * NP + nbase, NPT)],
                                tmp)

                def acc_l(i, _):
                    dacc[pl.ds(i * 16, 16)] += tmp[pl.ds(i * 16, 16)]
                    return 0
                lax.fori_loop(0, NPT // 16, acc_l, 0)
                return 0
            lax.fori_loop(0, NT, merge, 0)
            plsc.subcore_barrier()

    def mk_norm(i, _):
        do = jnp.maximum(no_a[pl.ds(i * 16, 16)], 1.0)
        di = jnp.maximum(ni_a[pl.ds(i * 16, 16)], 1.0)
        no = _rsqrt16(do)
        ni = _rsqrt16(di)
        no_a[pl.ds(i * 16, 16)] = no
        ni_a[pl.ds(i * 16, 16)] = ni
        a_a[pl.ds(i * 16, 16)] = (1.0 - ALPHA) * no * ni
        return 0
    lax.fori_loop(0, NARR // 16, mk_norm, 0)

    def _zero_edge_rows():
        def zr(r, _):
            for v in range(PW // 16):
                buf[r, pl.ds(v * 16, 16)] = zero16
            return 0
        lax.fori_loop(0, NRC, zr, 0)

    def init_node(q, _):
        r0 = nbase + q * NRC
        pltpu.sync_copy(feat_hbm.at[cid, pl.ds(r0, NRC)],
                        buf.at[pl.ds(NRC, NRC)])

        def initrow(r, _):
            nov = jnp.full((16,), no_a[pl.ds(q * NRC + r, 16)][0],
                           jnp.float32)
            for v in range(PW // 16):
                g0 = nov * buf[NRC + r, pl.ds(v * 16, 16)]
                buf[r, pl.ds(v * 16, 16)] = g0
                buf[NRC + r, pl.ds(v * 16, 16)] = ALPHA * g0
            return 0
        lax.fori_loop(0, NRC, initrow, 0)
        pltpu.sync_copy(buf.at[pl.ds(0, NRC)],
                        g_hbm.at[pl.ds(gplane + r0, NRC)])
        pltpu.sync_copy(buf.at[pl.ds(NRC, NRC)],
                        c0_hbm.at[cid, pl.ds(r0, NRC)])
        _zero_edge_rows()
        pltpu.sync_copy(buf.at[pl.ds(0, NRC)], acc.at[pl.ds(r0, NRC)])
        return 0
    lax.fori_loop(0, NNC, init_node, 0)
    plsc.subcore_barrier()

    NCNP = NC * NP

    def do_edges(par):
        offv = jnp.full((16,), par * NCNP + gplane, jnp.int32)

        def pair(jj, _):
            e0 = ebase + (2 * jj) * ECH
            pltpu.sync_copy(eidx_hbm.at[pl.ds(e0, ECH)], gidx_a)
            for k in range(ECH // 16):
                gidx_a[pl.ds(k * 16, 16)] += offv
            ga_d = pltpu.async_copy(g_hbm.at[gidx_a], buf, gsem_a)
            pltpu.sync_copy(eidx_hbm.at[pl.ds(e0 + ECH, ECH)], gidx_b)
            for k in range(ECH // 16):
                gidx_b[pl.ds(k * 16, 16)] += offv
            gb_d = pltpu.async_copy(g_hbm.at[gidx_b], buf_b, gsem_b)
            pltpu.sync_copy(eidx_hbm.at[pl.ds(EP + e0, ECH)], sidx_a)
            pltpu.sync_copy(eidx_hbm.at[pl.ds(EP + e0 + ECH, ECH)], sidx_b)
            ga_d.wait()
            sa_d = pltpu.async_copy(buf, acc.at[sidx_a], ssem, add=True)
            gb_d.wait()
            sb_d = pltpu.async_copy(buf_b, acc.at[sidx_b], ssem, add=True)
            sa_d.wait()
            sb_d.wait()
            return 0
        lax.fori_loop(0, NCH // 2, pair, 0)

    def step_body(t, _):
        par = lax.rem(t, 2)
        do_edges(par)
        plsc.subcore_barrier()

        def node(q, _):
            r0 = nbase + q * NRC
            pltpu.sync_copy(acc.at[pl.ds(r0, NRC)], buf.at[pl.ds(0, NRC)])
            pltpu.sync_copy(c0_hbm.at[cid, pl.ds(r0, NRC)],
                            buf.at[pl.ds(NRC, NRC)])

            def uprow(r, _):
                sa = jnp.full((16,), a_a[pl.ds(q * NRC + r, 16)][0],
                              jnp.float32)
                for v in range(PW // 16):
                    gv = sa * buf[r, pl.ds(v * 16, 16)]
                    gv = gv + buf[NRC + r, pl.ds(v * 16, 16)]
                    buf[r, pl.ds(v * 16, 16)] = gv
                return 0
            lax.fori_loop(0, NRC, uprow, 0)
            pltpu.sync_copy(
                buf.at[pl.ds(0, NRC)],
                g_hbm.at[pl.ds((1 - par) * NCNP + gplane + r0, NRC)])
            _zero_edge_rows()
            pltpu.sync_copy(buf.at[pl.ds(0, NRC)], acc.at[pl.ds(r0, NRC)])
            return 0
        lax.fori_loop(0, NNC, node, 0)
        plsc.subcore_barrier()
        return 0
    lax.fori_loop(0, KSTEPS - 1, step_body, 0)

    do_edges(jnp.int32((KSTEPS - 1) % 2))
    plsc.subcore_barrier()

    def node_last(q, _):
        r0 = nbase + q * NRC
        pltpu.sync_copy(acc.at[pl.ds(r0, NRC)], buf.at[pl.ds(0, NRC)])
        pltpu.sync_copy(feat_hbm.at[cid, pl.ds(r0, NRC)],
                        buf.at[pl.ds(NRC, NRC)])

        def uprow(r, _):
            sa = jnp.full((16,),
                          (1.0 - ALPHA) * ni_a[pl.ds(q * NRC + r, 16)][0],
                          jnp.float32)
            for v in range(PW // 16):
                gv = sa * buf[r, pl.ds(v * 16, 16)]
                gv = gv + ALPHA * buf[NRC + r, pl.ds(v * 16, 16)]
                buf[r, pl.ds(v * 16, 16)] = gv
            return 0
        lax.fori_loop(0, NRC, uprow, 0)
        pltpu.sync_copy(buf.at[pl.ds(0, NRC)],
                        hk_hbm.at[cid, pl.ds(r0, NRC)])
        return 0
    lax.fori_loop(0, NNC, node_last, 0)


_sc_propagate = pl.kernel(
    _sc_body,
    out_type=(jax.ShapeDtypeStruct((NC, NP, PW), jnp.float32),
              jax.ShapeDtypeStruct((NC, NP, PW), jnp.float32),
              jax.ShapeDtypeStruct((2 * NC * NP, PW), jnp.float32),
              jax.ShapeDtypeStruct((NC, NT * NP), jnp.float32)),
    mesh=plsc.VectorSubcoreMesh(core_axis_name="c", subcore_axis_name="s"),
    compiler_params=pltpu.CompilerParams(needs_layout_passes=False),
    scratch_types=[
        pltpu.VMEM_SHARED((NP, PW), jnp.float32),
        pltpu.VMEM((ECH,), jnp.int32),
        pltpu.VMEM((ECH,), jnp.int32),
        pltpu.VMEM((ECH,), jnp.int32),
        pltpu.VMEM((ECH,), jnp.int32),
        pltpu.VMEM((NP,), jnp.float32),
        pltpu.VMEM((2 * NRC, PW), jnp.float32),
        pltpu.VMEM((ECH, PW), jnp.float32),
        pltpu.VMEM((NPT,), jnp.float32),
        pltpu.VMEM((NARR,), jnp.float32),
        pltpu.VMEM((NARR,), jnp.float32),
        pltpu.VMEM((NARR,), jnp.float32),
        pltpu.SemaphoreType.DMA,
        pltpu.SemaphoreType.DMA,
        pltpu.SemaphoreType.DMA,
    ],
)


BN = 1000


def _tc_body(h_ref, w_ref, b_ref, a_ref, out_ref, gs_ref):
    i = pl.program_id(0)
    x = h_ref[...]
    y = lax.dot_general(x, w_ref[...], (((1,), (1,)), ((), ())),
                        preferred_element_type=jnp.float32)
    y = y + b_ref[...]
    a = a_ref[0, 0]
    y = jnp.where(y >= 0.0, y, a * y)
    out_ref[...] = y
    part = jnp.sum(y, axis=0, keepdims=True)

    @pl.when(i == 0)
    def _():
        gs_ref[...] = jnp.zeros((1, D), jnp.float32)
    gs_ref[...] += part


def _tc_head(hk, W, b, prelu_a):
    return pl.pallas_call(
        _tc_body,
        grid=(N // BN,),
        in_specs=[pl.BlockSpec((BN, D), lambda i: (i, 0)),
                  pl.BlockSpec((D, D), lambda i: (0, 0)),
                  pl.BlockSpec((1, D), lambda i: (0, 0)),
                  pl.BlockSpec((1, 1), lambda i: (0, 0))],
        out_specs=[pl.BlockSpec((BN, D), lambda i: (i, 0)),
                   pl.BlockSpec((1, D), lambda i: (0, 0))],
        out_shape=[jax.ShapeDtypeStruct((N, D), jnp.float32),
                   jax.ShapeDtypeStruct((1, D), jnp.float32)],
    )(hk, W, b.reshape(1, D), prelu_a.reshape(1, 1))


def kernel(feat, edge_index, W, b, prelu_a):
    feat_p = jnp.pad(feat, ((0, NP - N), (0, 0)))
    planes = [
        jnp.pad(feat_p[:, c * DH:(c + 1) * DH], ((0, 0), (0, PW - DH)))
        for c in range(NC)
    ]
    feat_cols = jnp.stack(planes)
    eidx_p = jnp.pad(edge_index, ((0, 0), (0, EP - E)), constant_values=N)
    hk_cols, _, _, _ = _sc_propagate(feat_cols, eidx_p.reshape(-1))
    hk = jnp.concatenate([hk_cols[c, :N, :DH] for c in range(NC)], axis=1)
    out, gsum = _tc_head(hk, W, b, prelu_a)
    return (out, gsum)


# speedup vs baseline: 1.8775x; 1.0184x over previous
import jax
import jax.numpy as jnp
from jax import lax
from jax.experimental import pallas as pl
from jax.experimental.pallas import tpu as pltpu
from jax.experimental.pallas import tpu_sc as plsc

N = 10000
E = 320000
D = 128
KSTEPS = 10
ALPHA = 0.1

NT = 16
NC = 2
DH = D // NC
PW = D
NP = 10240
NPT = NP // NT
NARR = NPT + 16
ECH = 128
EPT = 20480
EP = NT * EPT
NCH = EPT // ECH
NRC = 64
NNC = NPT // NRC


def _rsqrt16(x):
    i = plsc.bitcast(x, jnp.int32)
    y = plsc.bitcast(jnp.int32(0x5F3759DF) - (i >> 1), jnp.float32)
    for _ in range(3):
        y = y * (1.5 - 0.5 * x * y * y)
    return y


def _sc_body(feat_hbm, eidx_hbm, hk_hbm, c0_hbm, g_hbm, degall,
             acc, gidx_a, gidx_b, sidx_a, sidx_b,
             gidx_a2, gidx_b2, sidx_a2, sidx_b2, counts,
             buf, buf_b, tmp, no_a, ni_a, a_a,
             gsem_a, gsem_b, ssem, isem):
    cid = lax.axis_index("c")
    sid = lax.axis_index("s")
    ebase = sid * EPT
    nbase = sid * NPT
    gplane = cid * NP
    coff16 = jnp.full((16,), gplane, jnp.int32)

    zero16 = jnp.zeros((16,), jnp.float32)
    one16 = jnp.ones((16,), jnp.float32)

    if True:
        for rnd, dacc in ((0, no_a), (1, ni_a)):
            def z_counts(i, _):
                counts[pl.ds(i * 16, 16)] = zero16
                return 0
            lax.fori_loop(0, NP // 16, z_counts, 0)

            def cnt(j, _, rnd=rnd):
                pltpu.sync_copy(
                    eidx_hbm.at[pl.ds(rnd * EP + ebase + j * ECH, ECH)],
                    gidx_a)
                for k in range(ECH // 16):
                    ids = gidx_a[pl.ds(k * 16, 16)]
                    plsc.addupdate_scatter(counts, [ids], one16)
                return 0
            lax.fori_loop(0, NCH, cnt, 0)

            pltpu.sync_copy(counts, degall.at[cid, pl.ds(sid * NP, NP)])
            plsc.subcore_barrier()

            def z_deg(i, _, dacc=dacc):
                dacc[pl.ds(i * 16, 16)] = zero16
                return 0
            lax.fori_loop(0, NARR // 16, z_deg, 0)

            def merge(t2, _, dacc=dacc):
                pltpu.sync_copy(degall.at[cid, pl.ds(t2 * NP + nbase, NPT)],
                                tmp)

                def acc_l(i, _):
                    dacc[pl.ds(i * 16, 16)] += tmp[pl.ds(i * 16, 16)]
                    return 0
                lax.fori_loop(0, NPT // 16, acc_l, 0)
                return 0
            lax.fori_loop(0, NT, merge, 0)
            plsc.subcore_barrier()

    def mk_norm(i, _):
        do = jnp.maximum(no_a[pl.ds(i * 16, 16)], 1.0)
        di = jnp.maximum(ni_a[pl.ds(i * 16, 16)], 1.0)
        no = _rsqrt16(do)
        ni = _rsqrt16(di)
        no_a[pl.ds(i * 16, 16)] = no
        ni_a[pl.ds(i * 16, 16)] = ni
        a_a[pl.ds(i * 16, 16)] = (1.0 - ALPHA) * no * ni
        return 0
    lax.fori_loop(0, NARR // 16, mk_norm, 0)

    def _zero_edge_rows():
        def zr(r, _):
            for v in range(PW // 16):
                buf[r, pl.ds(v * 16, 16)] = zero16
            return 0
        lax.fori_loop(0, NRC, zr, 0)

    def init_node(q, _):
        r0 = nbase + q * NRC
        pltpu.sync_copy(feat_hbm.at[cid, pl.ds(r0, NRC)],
                        buf.at[pl.ds(NRC, NRC)])

        def initrow(r, _):
            nov = jnp.full((16,), no_a[pl.ds(q * NRC + r, 16)][0],
                           jnp.float32)
            for v in range(PW // 16):
                g0 = nov * buf[NRC + r, pl.ds(v * 16, 16)]
                buf[r, pl.ds(v * 16, 16)] = g0
                buf[NRC + r, pl.ds(v * 16, 16)] = ALPHA * g0
            return 0
        lax.fori_loop(0, NRC, initrow, 0)
        pltpu.sync_copy(buf.at[pl.ds(0, NRC)],
                        g_hbm.at[pl.ds(gplane + r0, NRC)])
        pltpu.sync_copy(buf.at[pl.ds(NRC, NRC)],
                        c0_hbm.at[cid, pl.ds(r0, NRC)])
        _zero_edge_rows()
        pltpu.sync_copy(buf.at[pl.ds(0, NRC)], acc.at[pl.ds(r0, NRC)])
        return 0
    lax.fori_loop(0, NNC, init_node, 0)
    plsc.subcore_barrier()

    NCNP = NC * NP

    def do_edges(par):
        offv = jnp.full((16,), par * NCNP + gplane, jnp.int32)
        NPAIR = NCH // 2

        def load_idx(p, gi, si, gj, sj):
            e0 = ebase + (2 * p) * ECH
            return (pltpu.async_copy(eidx_hbm.at[pl.ds(e0, ECH)], gi, isem),
                    pltpu.async_copy(eidx_hbm.at[pl.ds(EP + e0, ECH)], si,
                                     isem),
                    pltpu.async_copy(eidx_hbm.at[pl.ds(e0 + ECH, ECH)], gj,
                                     isem),
                    pltpu.async_copy(eidx_hbm.at[pl.ds(EP + e0 + ECH, ECH)],
                                     sj, isem))

        def proc_pair(gi, si, gj, sj):
            for k in range(ECH // 16):
                gi[pl.ds(k * 16, 16)] += offv
            ga_d = pltpu.async_copy(g_hbm.at[gi], buf, gsem_a)
            for k in range(ECH // 16):
                gj[pl.ds(k * 16, 16)] += offv
            gb_d = pltpu.async_copy(g_hbm.at[gj], buf_b, gsem_b)
            ga_d.wait()
            sa_d = pltpu.async_copy(buf, acc.at[si], ssem, add=True)
            gb_d.wait()
            sb_d = pltpu.async_copy(buf_b, acc.at[sj], ssem, add=True)
            sa_d.wait()
            sb_d.wait()

        for d in load_idx(0, gidx_a, sidx_a, gidx_b, sidx_b):
            d.wait()

        def pairs2(u, _):
            p1 = 2 * u + 1
            p2 = jnp.minimum(2 * u + 2, NPAIR - 1)
            ds1 = load_idx(p1, gidx_a2, sidx_a2, gidx_b2, sidx_b2)
            proc_pair(gidx_a, sidx_a, gidx_b, sidx_b)
            ds2 = load_idx(p2, gidx_a, sidx_a, gidx_b, sidx_b)
            for d in ds1:
                d.wait()
            proc_pair(gidx_a2, sidx_a2, gidx_b2, sidx_b2)
            for d in ds2:
                d.wait()
            return 0
        lax.fori_loop(0, NPAIR // 2, pairs2, 0)

    def step_body(t, _):
        par = lax.rem(t, 2)
        do_edges(par)
        plsc.subcore_barrier()

        def node(q, _):
            r0 = nbase + q * NRC
            pltpu.sync_copy(acc.at[pl.ds(r0, NRC)], buf.at[pl.ds(0, NRC)])
            pltpu.sync_copy(c0_hbm.at[cid, pl.ds(r0, NRC)],
                            buf.at[pl.ds(NRC, NRC)])

            def uprow(r, _):
                sa = jnp.full((16,), a_a[pl.ds(q * NRC + r, 16)][0],
                              jnp.float32)
                for v in range(PW // 16):
                    gv = sa * buf[r, pl.ds(v * 16, 16)]
                    gv = gv + buf[NRC + r, pl.ds(v * 16, 16)]
                    buf[r, pl.ds(v * 16, 16)] = gv
                return 0
            lax.fori_loop(0, NRC, uprow, 0)
            pltpu.sync_copy(
                buf.at[pl.ds(0, NRC)],
                g_hbm.at[pl.ds((1 - par) * NCNP + gplane + r0, NRC)])
            _zero_edge_rows()
            pltpu.sync_copy(buf.at[pl.ds(0, NRC)], acc.at[pl.ds(r0, NRC)])
            return 0
        lax.fori_loop(0, NNC, node, 0)
        plsc.subcore_barrier()
        return 0
    lax.fori_loop(0, KSTEPS - 1, step_body, 0)

    do_edges(jnp.int32((KSTEPS - 1) % 2))
    plsc.subcore_barrier()

    def node_last(q, _):
        r0 = nbase + q * NRC
        pltpu.sync_copy(acc.at[pl.ds(r0, NRC)], buf.at[pl.ds(0, NRC)])
        pltpu.sync_copy(feat_hbm.at[cid, pl.ds(r0, NRC)],
                        buf.at[pl.ds(NRC, NRC)])

        def uprow(r, _):
            sa = jnp.full((16,),
                          (1.0 - ALPHA) * ni_a[pl.ds(q * NRC + r, 16)][0],
                          jnp.float32)
            for v in range(PW // 16):
                gv = sa * buf[r, pl.ds(v * 16, 16)]
                gv = gv + ALPHA * buf[NRC + r, pl.ds(v * 16, 16)]
                buf[r, pl.ds(v * 16, 16)] = gv
            return 0
        lax.fori_loop(0, NRC, uprow, 0)
        pltpu.sync_copy(buf.at[pl.ds(0, NRC)],
                        hk_hbm.at[cid, pl.ds(r0, NRC)])
        return 0
    lax.fori_loop(0, NNC, node_last, 0)


_sc_propagate = pl.kernel(
    _sc_body,
    out_type=(jax.ShapeDtypeStruct((NC, NP, PW), jnp.float32),
              jax.ShapeDtypeStruct((NC, NP, PW), jnp.float32),
              jax.ShapeDtypeStruct((2 * NC * NP, PW), jnp.float32),
              jax.ShapeDtypeStruct((NC, NT * NP), jnp.float32)),
    mesh=plsc.VectorSubcoreMesh(core_axis_name="c", subcore_axis_name="s"),
    compiler_params=pltpu.CompilerParams(needs_layout_passes=False),
    scratch_types=[
        pltpu.VMEM_SHARED((NP, PW), jnp.float32),
        pltpu.VMEM((ECH,), jnp.int32),
        pltpu.VMEM((ECH,), jnp.int32),
        pltpu.VMEM((ECH,), jnp.int32),
        pltpu.VMEM((ECH,), jnp.int32),
        pltpu.VMEM((ECH,), jnp.int32),
        pltpu.VMEM((ECH,), jnp.int32),
        pltpu.VMEM((ECH,), jnp.int32),
        pltpu.VMEM((ECH,), jnp.int32),
        pltpu.VMEM((NP,), jnp.float32),
        pltpu.VMEM((2 * NRC, PW), jnp.float32),
        pltpu.VMEM((ECH, PW), jnp.float32),
        pltpu.VMEM((NPT,), jnp.float32),
        pltpu.VMEM((NARR,), jnp.float32),
        pltpu.VMEM((NARR,), jnp.float32),
        pltpu.VMEM((NARR,), jnp.float32),
        pltpu.SemaphoreType.DMA,
        pltpu.SemaphoreType.DMA,
        pltpu.SemaphoreType.DMA,
        pltpu.SemaphoreType.DMA,
    ],
)


BN = 1000


def _tc_body(h_ref, w_ref, b_ref, a_ref, out_ref, gs_ref):
    i = pl.program_id(0)
    x = h_ref[...]
    y = lax.dot_general(x, w_ref[...], (((1,), (1,)), ((), ())),
                        preferred_element_type=jnp.float32)
    y = y + b_ref[...]
    a = a_ref[0, 0]
    y = jnp.where(y >= 0.0, y, a * y)
    out_ref[...] = y
    part = jnp.sum(y, axis=0, keepdims=True)

    @pl.when(i == 0)
    def _():
        gs_ref[...] = jnp.zeros((1, D), jnp.float32)
    gs_ref[...] += part


def _tc_head(hk, W, b, prelu_a):
    return pl.pallas_call(
        _tc_body,
        grid=(N // BN,),
        in_specs=[pl.BlockSpec((BN, D), lambda i: (i, 0)),
                  pl.BlockSpec((D, D), lambda i: (0, 0)),
                  pl.BlockSpec((1, D), lambda i: (0, 0)),
                  pl.BlockSpec((1, 1), lambda i: (0, 0))],
        out_specs=[pl.BlockSpec((BN, D), lambda i: (i, 0)),
                   pl.BlockSpec((1, D), lambda i: (0, 0))],
        out_shape=[jax.ShapeDtypeStruct((N, D), jnp.float32),
                   jax.ShapeDtypeStruct((1, D), jnp.float32)],
    )(hk, W, b.reshape(1, D), prelu_a.reshape(1, 1))


def kernel(feat, edge_index, W, b, prelu_a):
    feat_p = jnp.pad(feat, ((0, NP - N), (0, 0)))
    planes = [
        jnp.pad(feat_p[:, c * DH:(c + 1) * DH], ((0, 0), (0, PW - DH)))
        for c in range(NC)
    ]
    feat_cols = jnp.stack(planes)
    eidx_p = jnp.pad(edge_index, ((0, 0), (0, EP - E)), constant_values=N)
    hk_cols, _, _, _ = _sc_propagate(feat_cols, eidx_p.reshape(-1))
    hk = jnp.concatenate([hk_cols[c, :N, :DH] for c in range(NC)], axis=1)
    out, gsum = _tc_head(hk, W, b, prelu_a)
    return (out, gsum)


# speedup vs baseline: 1.8787x; 1.0006x over previous
import jax
import jax.numpy as jnp
from jax import lax
from jax.experimental import pallas as pl
from jax.experimental.pallas import tpu as pltpu
from jax.experimental.pallas import tpu_sc as plsc

N = 10000
E = 320000
D = 128
KSTEPS = 10
ALPHA = 0.1

NT = 16
NC = 2
DH = D // NC
PW = D
NP = 10240
NPT = NP // NT
NARR = NPT + 16
ECH = 64
EPT = 20480
EP = NT * EPT
NCH = EPT // ECH
NRC = 64
NNC = NPT // NRC


def _rsqrt16(x):
    i = plsc.bitcast(x, jnp.int32)
    y = plsc.bitcast(jnp.int32(0x5F3759DF) - (i >> 1), jnp.float32)
    for _ in range(3):
        y = y * (1.5 - 0.5 * x * y * y)
    return y


def _sc_body(feat_hbm, eidx_hbm, hk_hbm, c0_hbm, g_hbm, degall,
             acc, gidx_a, gidx_b, sidx_a, sidx_b,
             gidx_a2, gidx_b2, sidx_a2, sidx_b2, counts,
             buf, buf_b, buf_c, buf_d, tmp, no_a, ni_a, a_a,
             gsem_a, gsem_b, gsem_c, gsem_d, ssem, isem):
    cid = lax.axis_index("c")
    sid = lax.axis_index("s")
    ebase = sid * EPT
    nbase = sid * NPT
    gplane = cid * NP

    zero16 = jnp.zeros((16,), jnp.float32)
    one16 = jnp.ones((16,), jnp.float32)

    if True:
        for rnd, dacc in ((0, no_a), (1, ni_a)):
            def z_counts(i, _):
                counts[pl.ds(i * 16, 16)] = zero16
                return 0
            lax.fori_loop(0, NP // 16, z_counts, 0)

            def cnt(j, _, rnd=rnd):
                pltpu.sync_copy(
                    eidx_hbm.at[pl.ds(rnd * EP + ebase + j * 2 * ECH,
                                      2 * ECH)],
                    gidx_a)
                for k in range((2 * ECH) // 16):
                    ids = gidx_a[pl.ds(k * 16, 16)]
                    plsc.addupdate_scatter(counts, [ids], one16)
                return 0
            lax.fori_loop(0, NCH // 2, cnt, 0)

            pltpu.sync_copy(counts, degall.at[cid, pl.ds(sid * NP, NP)])
            plsc.subcore_barrier()

            def z_deg(i, _, dacc=dacc):
                dacc[pl.ds(i * 16, 16)] = zero16
                return 0
            lax.fori_loop(0, NARR // 16, z_deg, 0)

            def merge(t2, _, dacc=dacc):
                pltpu.sync_copy(degall.at[cid, pl.ds(t2 * NP + nbase, NPT)],
                                tmp)

                def acc_l(i, _):
                    dacc[pl.ds(i * 16, 16)] += tmp[pl.ds(i * 16, 16)]
                    return 0
                lax.fori_loop(0, NPT // 16, acc_l, 0)
                return 0
            lax.fori_loop(0, NT, merge, 0)
            plsc.subcore_barrier()

    def mk_norm(i, _):
        do = jnp.maximum(no_a[pl.ds(i * 16, 16)], 1.0)
        di = jnp.maximum(ni_a[pl.ds(i * 16, 16)], 1.0)
        no = _rsqrt16(do)
        ni = _rsqrt16(di)
        no_a[pl.ds(i * 16, 16)] = no
        ni_a[pl.ds(i * 16, 16)] = ni
        a_a[pl.ds(i * 16, 16)] = (1.0 - ALPHA) * no * ni
        return 0
    lax.fori_loop(0, NARR // 16, mk_norm, 0)

    def _zero_rows(bref):
        def zr(r, _):
            for v in range(PW // 16):
                bref[r, pl.ds(v * 16, 16)] = zero16
            return 0
        lax.fori_loop(0, NRC, zr, 0)

    def init_node(q, _):
        r0 = nbase + q * NRC
        pltpu.sync_copy(feat_hbm.at[cid, pl.ds(r0, NRC)],
                        buf_b.at[pl.ds(0, NRC)])

        def initrow(r, _):
            nov = jnp.full((16,), no_a[pl.ds(q * NRC + r, 16)][0],
                           jnp.float32)
            for v in range(PW // 16):
                g0 = nov * buf_b[r, pl.ds(v * 16, 16)]
                buf[r, pl.ds(v * 16, 16)] = g0
                buf_b[r, pl.ds(v * 16, 16)] = ALPHA * g0
            return 0
        lax.fori_loop(0, NRC, initrow, 0)
        pltpu.sync_copy(buf.at[pl.ds(0, NRC)],
                        g_hbm.at[pl.ds(gplane + r0, NRC)])
        pltpu.sync_copy(buf_b.at[pl.ds(0, NRC)],
                        c0_hbm.at[cid, pl.ds(r0, NRC)])
        _zero_rows(buf)
        pltpu.sync_copy(buf.at[pl.ds(0, NRC)], acc.at[pl.ds(r0, NRC)])
        return 0
    lax.fori_loop(0, NNC, init_node, 0)
    plsc.subcore_barrier()

    NCNP = NC * NP

    def do_edges(par):
        offg = jnp.full((16,), par * NCNP + gplane, jnp.int32)
        NQUAD = NCH // 4
        bufs = (buf, buf_b, buf_c, buf_d)
        gsems = (gsem_a, gsem_b, gsem_c, gsem_d)
        set0 = (gidx_a, sidx_a, gidx_b, sidx_b)
        set1 = (gidx_a2, sidx_a2, gidx_b2, sidx_b2)

        def load_idx(qd, st):
            gi, si, gj, sj = st
            e0 = ebase + (4 * qd) * ECH
            return (
                pltpu.async_copy(eidx_hbm.at[pl.ds(e0, 2 * ECH)], gi, isem),
                pltpu.async_copy(eidx_hbm.at[pl.ds(EP + e0, 2 * ECH)], si,
                                 isem),
                pltpu.async_copy(eidx_hbm.at[pl.ds(e0 + 2 * ECH, 2 * ECH)],
                                 gj, isem),
                pltpu.async_copy(
                    eidx_hbm.at[pl.ds(EP + e0 + 2 * ECH, 2 * ECH)], sj,
                    isem))

        def proc_quad(st):
            gi, si, gj, sj = st
            for k in range((2 * ECH) // 16):
                gi[pl.ds(k * 16, 16)] += offg
                gj[pl.ds(k * 16, 16)] += offg
            g_ds = (
                pltpu.async_copy(g_hbm.at[gi.at[pl.ds(0, ECH)]], buf,
                                 gsem_a),
                pltpu.async_copy(g_hbm.at[gi.at[pl.ds(ECH, ECH)]], buf_b,
                                 gsem_b),
                pltpu.async_copy(g_hbm.at[gj.at[pl.ds(0, ECH)]], buf_c,
                                 gsem_c),
                pltpu.async_copy(g_hbm.at[gj.at[pl.ds(ECH, ECH)]], buf_d,
                                 gsem_d),
            )
            s_ds = []
            for i, (sref, soff) in enumerate(
                    ((si, 0), (si, ECH), (sj, 0), (sj, ECH))):
                g_ds[i].wait()
                s_ds.append(pltpu.async_copy(
                    bufs[i], acc.at[sref.at[pl.ds(soff, ECH)]], ssem,
                    add=True))
            for d in s_ds:
                d.wait()

        for d in load_idx(0, set0):
            d.wait()

        def quads2(u, _):
            q1 = 2 * u + 1
            q2 = jnp.minimum(2 * u + 2, NQUAD - 1)
            ds1 = load_idx(q1, set1)
            proc_quad(set0)
            ds2 = load_idx(q2, set0)
            for d in ds1:
                d.wait()
            proc_quad(set1)
            for d in ds2:
                d.wait()
            return 0
        lax.fori_loop(0, NQUAD // 2, quads2, 0)

    def step_body(t, _):
        par = lax.rem(t, 2)
        do_edges(par)
        plsc.subcore_barrier()

        def node(q, _):
            r0 = nbase + q * NRC
            pltpu.sync_copy(acc.at[pl.ds(r0, NRC)], buf.at[pl.ds(0, NRC)])
            pltpu.sync_copy(c0_hbm.at[cid, pl.ds(r0, NRC)],
                            buf_b.at[pl.ds(0, NRC)])

            def uprow(r, _):
                sa = jnp.full((16,), a_a[pl.ds(q * NRC + r, 16)][0],
                              jnp.float32)
                for v in range(PW // 16):
                    gv = sa * buf[r, pl.ds(v * 16, 16)]
                    gv = gv + buf_b[r, pl.ds(v * 16, 16)]
                    buf[r, pl.ds(v * 16, 16)] = gv
                return 0
            lax.fori_loop(0, NRC, uprow, 0)
            pltpu.sync_copy(
                buf.at[pl.ds(0, NRC)],
                g_hbm.at[pl.ds((1 - par) * NCNP + gplane + r0, NRC)])
            _zero_rows(buf)
            pltpu.sync_copy(buf.at[pl.ds(0, NRC)], acc.at[pl.ds(r0, NRC)])
            return 0
        lax.fori_loop(0, NNC, node, 0)
        plsc.subcore_barrier()
        return 0
    lax.fori_loop(0, KSTEPS - 1, step_body, 0)

    do_edges(jnp.int32((KSTEPS - 1) % 2))
    plsc.subcore_barrier()

    def node_last(q, _):
        r0 = nbase + q * NRC
        pltpu.sync_copy(acc.at[pl.ds(r0, NRC)], buf.at[pl.ds(0, NRC)])
        pltpu.sync_copy(feat_hbm.at[cid, pl.ds(r0, NRC)],
                        buf_b.at[pl.ds(0, NRC)])

        def uprow(r, _):
            sa = jnp.full((16,),
                          (1.0 - ALPHA) * ni_a[pl.ds(q * NRC + r, 16)][0],
                          jnp.float32)
            for v in range(PW // 16):
                gv = sa * buf[r, pl.ds(v * 16, 16)]
                gv = gv + ALPHA * buf_b[r, pl.ds(v * 16, 16)]
                buf[r, pl.ds(v * 16, 16)] = gv
            return 0
        lax.fori_loop(0, NRC, uprow, 0)
        pltpu.sync_copy(buf.at[pl.ds(0, NRC)],
                        hk_hbm.at[cid, pl.ds(r0, NRC)])
        return 0
    lax.fori_loop(0, NNC, node_last, 0)


_sc_propagate = pl.kernel(
    _sc_body,
    out_type=(jax.ShapeDtypeStruct((NC, NP, PW), jnp.float32),
              jax.ShapeDtypeStruct((NC, NP, PW), jnp.float32),
              jax.ShapeDtypeStruct((2 * NC * NP, PW), jnp.float32),
              jax.ShapeDtypeStruct((NC, NT * NP), jnp.float32)),
    mesh=plsc.VectorSubcoreMesh(core_axis_name="c", subcore_axis_name="s"),
    compiler_params=pltpu.CompilerParams(needs_layout_passes=False),
    scratch_types=[
        pltpu.VMEM_SHARED((NP, PW), jnp.float32),
        pltpu.VMEM((2 * ECH,), jnp.int32),
        pltpu.VMEM((2 * ECH,), jnp.int32),
        pltpu.VMEM((2 * ECH,), jnp.int32),
        pltpu.VMEM((2 * ECH,), jnp.int32),
        pltpu.VMEM((2 * ECH,), jnp.int32),
        pltpu.VMEM((2 * ECH,), jnp.int32),
        pltpu.VMEM((2 * ECH,), jnp.int32),
        pltpu.VMEM((2 * ECH,), jnp.int32),
        pltpu.VMEM((NP,), jnp.float32),
        pltpu.VMEM((ECH, PW), jnp.float32),
        pltpu.VMEM((ECH, PW), jnp.float32),
        pltpu.VMEM((ECH, PW), jnp.float32),
        pltpu.VMEM((ECH, PW), jnp.float32),
        pltpu.VMEM((NPT,), jnp.float32),
        pltpu.VMEM((NARR,), jnp.float32),
        pltpu.VMEM((NARR,), jnp.float32),
        pltpu.VMEM((NARR,), jnp.float32),
        pltpu.SemaphoreType.DMA,
        pltpu.SemaphoreType.DMA,
        pltpu.SemaphoreType.DMA,
        pltpu.SemaphoreType.DMA,
        pltpu.SemaphoreType.DMA,
        pltpu.SemaphoreType.DMA,
    ],
)


BN = 1000


def _tc_body(h_ref, w_ref, b_ref, a_ref, out_ref, gs_ref):
    i = pl.program_id(0)
    x = h_ref[...]
    y = lax.dot_general(x, w_ref[...], (((1,), (1,)), ((), ())),
                        preferred_element_type=jnp.float32)
    y = y + b_ref[...]
    a = a_ref[0, 0]
    y = jnp.where(y >= 0.0, y, a * y)
    out_ref[...] = y
    part = jnp.sum(y, axis=0, keepdims=True)

    @pl.when(i == 0)
    def _():
        gs_ref[...] = jnp.zeros((1, D), jnp.float32)
    gs_ref[...] += part


def _tc_head(hk, W, b, prelu_a):
    return pl.pallas_call(
        _tc_body,
        grid=(N // BN,),
        in_specs=[pl.BlockSpec((BN, D), lambda i: (i, 0)),
                  pl.BlockSpec((D, D), lambda i: (0, 0)),
                  pl.BlockSpec((1, D), lambda i: (0, 0)),
                  pl.BlockSpec((1, 1), lambda i: (0, 0))],
        out_specs=[pl.BlockSpec((BN, D), lambda i: (i, 0)),
                   pl.BlockSpec((1, D), lambda i: (0, 0))],
        out_shape=[jax.ShapeDtypeStruct((N, D), jnp.float32),
                   jax.ShapeDtypeStruct((1, D), jnp.float32)],
    )(hk, W, b.reshape(1, D), prelu_a.reshape(1, 1))


def kernel(feat, edge_index, W, b, prelu_a):
    feat_p = jnp.pad(feat, ((0, NP - N), (0, 0)))
    feat_cols = jnp.stack([
        jnp.pad(feat_p[:, c * DH:(c + 1) * DH], ((0, 0), (0, PW - DH)))
        for c in range(NC)
    ])
    eidx_p = jnp.pad(edge_index, ((0, 0), (0, EP - E)), constant_values=N)
    hk_cols, _, _, _ = _sc_propagate(feat_cols, eidx_p.reshape(-1))
    hk = jnp.concatenate([hk_cols[c, :N, :DH] for c in range(NC)], axis=1)
    out, gsum = _tc_head(hk, W, b, prelu_a)
    return (out, gsum)
